# Initial kernel scaffold; baseline (speedup 1.0000x reference)
import jax
import jax.numpy as jnp
from jax.experimental import pallas as pl


def kernel(x, batch):
    raise NotImplementedError("write your pallas kernel here")


# speedup vs baseline: 4.0746x; 4.0746x over previous
import functools

import jax
import jax.numpy as jnp
from jax import lax
from jax.experimental import pallas as pl
from jax.experimental.pallas import tpu as pltpu
from jax.experimental.pallas import tpu_sc as plsc

N, D, S = 50000, 512, 64
NC, NS = 2, 16
NW = NC * NS
CH = 80
TB = 20
TS = 19
BIGW = 17
RB = CH * TB
RS = CH * TS
DV = D // 16
CW = 16


def _sc_segment_sums(xf, batch):
    mesh = plsc.VectorSubcoreMesh(core_axis_name="c", subcore_axis_name="s")

    @functools.partial(
        pl.kernel,
        mesh=mesh,
        out_type=[
            jax.ShapeDtypeStruct((NW, S * D), jnp.float32),
            jax.ShapeDtypeStruct((NW, S * CW), jnp.float32),
        ],
        scratch_types=[
            pltpu.VMEM((CH * D,), jnp.float32),
            pltpu.VMEM((CH * D,), jnp.float32),
            pltpu.VMEM((RB + 16,), jnp.int32),
            pltpu.VMEM((S * D,), jnp.float32),
            pltpu.VMEM((S * CW,), jnp.float32),
            pltpu.SemaphoreType.DMA,
            pltpu.SemaphoreType.DMA,
        ],
    )
    def seg(x_hbm, b_hbm, sums_hbm, cnts_hbm, rows0, rows1, idxv, acc, cntv,
            sem0, sem1):
        cid = lax.axis_index("c")
        sid = lax.axis_index("s")
        wid = cid * NS + sid
        big = wid < BIGW
        wbase = jnp.where(big, wid * RB, BIGW * RB + (wid - BIGW) * RS)

        zv = jnp.zeros((16,), jnp.float32)

        def zero_body(s, _):
            for j in range(DV):
                acc[pl.ds(s * D + 16 * j, 16)] = zv
            return 0

        lax.fori_loop(0, S, zero_body, 0)
        for k in range(S * CW // 16):
            cntv[pl.ds(16 * k, 16)] = zv

        @pl.when(big)
        def _():
            pltpu.sync_copy(b_hbm.at[pl.ds(wbase, RB)], idxv.at[pl.ds(0, RB)])

        @pl.when(jnp.logical_not(big))
        def _():
            pltpu.sync_copy(b_hbm.at[pl.ds(wbase, RS)], idxv.at[pl.ds(0, RS)])

        sems = [sem0, sem1]
        bufs = [rows0, rows1]

        def issue(t):
            pltpu.async_copy(x_hbm.at[pl.ds((wbase + t * CH) * D, CH * D)],
                             bufs[t % 2], sems[t % 2])

        def wait_t(t):
            pltpu.make_async_copy(x_hbm.at[pl.ds((wbase + t * CH) * D, CH * D)],
                                  bufs[t % 2], sems[t % 2]).wait()

        def make_row_body(t):
            buf = bufs[t % 2]

            def row_body(r, carry):
                cur = carry[0]
                cnt = carry[1]
                sums = carry[2:]
                s = idxv[pl.ds(t * CH + r, 16)][0]
                bnd = s != cur

                @pl.when(bnd)
                def _():
                    for j in range(DV):
                        acc[pl.ds(cur * D + 16 * j, 16)] = sums[j]
                    cntv[pl.ds(cur * CW, CW)] = jnp.broadcast_to(cnt, (CW,))

                keep = jnp.where(bnd, 0.0, 1.0)
                new_sums = tuple(
                    sums[j] * keep + buf[pl.ds(r * D + 16 * j, 16)]
                    for j in range(DV))
                new_cnt = cnt * keep + 1.0
                return (s, new_cnt) + new_sums

            return row_body

        carry = (idxv[pl.ds(0, 16)][0], jnp.float32(0.0)) + tuple(
            jnp.zeros((16,), jnp.float32) for _ in range(DV))

        issue(0)
        for t in range(TB):
            if t < TB - 1:
                wait_t(t)
                if t + 1 == TB - 1:
                    @pl.when(big)
                    def _():
                        issue(TB - 1)
                else:
                    issue(t + 1)
                carry = lax.fori_loop(0, CH, make_row_body(t), carry)
            else:
                @pl.when(big)
                def _():
                    wait_t(TB - 1)
                nlast = jnp.where(big, CH, 0)
                carry = lax.fori_loop(0, nlast, make_row_body(t), carry)

        cur = carry[0]
        cnt = carry[1]
        sums = carry[2:]
        for j in range(DV):
            acc[pl.ds(cur * D + 16 * j, 16)] = sums[j]
        cntv[pl.ds(cur * CW, CW)] = jnp.broadcast_to(cnt, (CW,))

        pltpu.sync_copy(acc, sums_hbm.at[wid])
        pltpu.sync_copy(cntv, cnts_hbm.at[wid])

    return seg(xf, batch)


def _merge_kernel(s_ref, c_ref, o_ref):
    sums = jnp.sum(s_ref[...].reshape(NW, S, D), axis=0)
    cnt = jnp.sum(c_ref[...].reshape(NW, S, CW), axis=0)[:, 0:1]
    o_ref[...] = sums / jnp.maximum(cnt, 1.0)


def kernel(x, batch):
    sums, cnts = _sc_segment_sums(x.reshape(-1), batch)
    return pl.pallas_call(
        _merge_kernel,
        out_shape=jax.ShapeDtypeStruct((S, D), jnp.float32),
    )(sums, cnts)


# speedup vs baseline: 4.3664x; 1.0716x over previous
import functools

import jax
import jax.numpy as jnp
from jax import lax
from jax.experimental import pallas as pl
from jax.experimental.pallas import tpu as pltpu
from jax.experimental.pallas import tpu_sc as plsc

N, D, S = 50000, 512, 64
NC, NS = 2, 16
NW = NC * NS
CH = 80
NB = CH // 16
TB = 20
TS = 19
BIGW = 17
RB = CH * TB
RS = CH * TS
DV = D // 16
CW = 16
CHD = CH * D


def _sc_segment_sums(xf, batch):
    mesh = plsc.VectorSubcoreMesh(core_axis_name="c", subcore_axis_name="s")

    @functools.partial(
        pl.kernel,
        mesh=mesh,
        out_type=[
            jax.ShapeDtypeStruct((NW, S * D), jnp.float32),
            jax.ShapeDtypeStruct((NW, S * CW), jnp.float32),
        ],
        scratch_types=[
            pltpu.VMEM((2 * CHD,), jnp.float32),
            pltpu.VMEM((RB + 16,), jnp.int32),
            pltpu.VMEM((S * D,), jnp.float32),
            pltpu.VMEM((S * CW,), jnp.float32),
            pltpu.SemaphoreType.DMA,
            pltpu.SemaphoreType.DMA,
        ],
    )
    def seg(x_hbm, b_hbm, sums_hbm, cnts_hbm, rows, idxv, acc, cntv,
            sem0, sem1):
        cid = lax.axis_index("c")
        sid = lax.axis_index("s")
        wid = cid * NS + sid
        big = wid < BIGW
        nch = jnp.where(big, TB, TS)
        wbase = jnp.where(big, wid * RB, BIGW * RB + (wid - BIGW) * RS)

        zv = jnp.zeros((16,), jnp.float32)

        def zero_body(s, _):
            for j in range(DV):
                acc[pl.ds(s * D + 16 * j, 16)] = zv
            return 0

        lax.fori_loop(0, S, zero_body, 0)
        for k in range(S * CW // 16):
            cntv[pl.ds(16 * k, 16)] = zv

        @pl.when(big)
        def _():
            pltpu.sync_copy(b_hbm.at[pl.ds(wbase, RB)], idxv.at[pl.ds(0, RB)])

        @pl.when(jnp.logical_not(big))
        def _():
            pltpu.sync_copy(b_hbm.at[pl.ds(wbase, RS)], idxv.at[pl.ds(0, RS)])

        def issue(t):
            @pl.when(lax.rem(t, 2) == 0)
            def _():
                pltpu.async_copy(
                    x_hbm.at[pl.ds((wbase + t * CH) * D, CHD)],
                    rows.at[pl.ds(0, CHD)], sem0)

            @pl.when(lax.rem(t, 2) == 1)
            def _():
                pltpu.async_copy(
                    x_hbm.at[pl.ds((wbase + t * CH) * D, CHD)],
                    rows.at[pl.ds(CHD, CHD)], sem1)

        def wait_t(t):
            @pl.when(lax.rem(t, 2) == 0)
            def _():
                pltpu.make_async_copy(
                    x_hbm.at[pl.ds((wbase + t * CH) * D, CHD)],
                    rows.at[pl.ds(0, CHD)], sem0).wait()

            @pl.when(lax.rem(t, 2) == 1)
            def _():
                pltpu.make_async_copy(
                    x_hbm.at[pl.ds((wbase + t * CH) * D, CHD)],
                    rows.at[pl.ds(CHD, CHD)], sem1).wait()

        sixteen = jnp.full((CW,), 16.0, jnp.float32)
        one = jnp.ones((CW,), jnp.float32)

        def block_body(t, bk, _):
            g = t * CH + bk * 16
            base2 = lax.rem(t, 2) * CHD + bk * 16 * D
            bv = idxv[pl.ds(g, 16)]
            uniform = bv[0] == bv[15]

            @pl.when(uniform)
            def _():
                s = bv[0]
                for j in range(DV):
                    a = rows[pl.ds(base2 + 16 * j, 16)]
                    for r in range(1, 16):
                        a = a + rows[pl.ds(base2 + r * D + 16 * j, 16)]
                    plsc.addupdate(acc.at[pl.ds(s * D + 16 * j, 16)], a)
                plsc.addupdate(cntv.at[pl.ds(s * CW, CW)], sixteen)

            @pl.when(jnp.logical_not(uniform))
            def _():
                for r in range(16):
                    sr = bv[r]
                    for j in range(DV):
                        plsc.addupdate(
                            acc.at[pl.ds(sr * D + 16 * j, 16)],
                            rows[pl.ds(base2 + r * D + 16 * j, 16)])
                    plsc.addupdate(cntv.at[pl.ds(sr * CW, CW)], one)

            return 0

        def chunk_body(t, c):
            wait_t(t)

            @pl.when(t + 1 < nch)
            def _():
                issue(t + 1)

            return lax.fori_loop(0, NB, lambda bk, cc: block_body(t, bk, cc), c)

        issue(0)
        lax.fori_loop(0, nch, chunk_body, 0)

        pltpu.sync_copy(acc, sums_hbm.at[wid])
        pltpu.sync_copy(cntv, cnts_hbm.at[wid])

    return seg(xf, batch)


def _merge_kernel(s_ref, c_ref, o_ref):
    sums = jnp.sum(s_ref[...].reshape(NW, S, D), axis=0)
    cnt = jnp.sum(c_ref[...].reshape(NW, S, CW), axis=0)[:, 0:1]
    o_ref[...] = sums / jnp.maximum(cnt, 1.0)


def kernel(x, batch):
    sums, cnts = _sc_segment_sums(x.reshape(-1), batch)
    return pl.pallas_call(
        _merge_kernel,
        out_shape=jax.ShapeDtypeStruct((S, D), jnp.float32),
    )(sums, cnts)


# speedup vs baseline: 7.3949x; 1.6936x over previous
import functools

import jax
import jax.numpy as jnp
from jax import lax
from jax.experimental import pallas as pl
from jax.experimental.pallas import tpu as pltpu
from jax.experimental.pallas import tpu_sc as plsc

N, D, S = 50000, 512, 64
NC, NS = 2, 16
NW = NC * NS
CH = 80
NB = CH // 16
TB = 20
TS = 19
BIGW = 17
RB = CH * TB
RS = CH * TS
DV = D // 16
CW = 16
CHD = CH * D


def _sc_segment_sums(xf, batch):
    mesh = plsc.VectorSubcoreMesh(core_axis_name="c", subcore_axis_name="s")

    @functools.partial(
        pl.kernel,
        mesh=mesh,
        out_type=[
            jax.ShapeDtypeStruct((NW, S * D), jnp.float32),
            jax.ShapeDtypeStruct((NW, S * CW), jnp.float32),
        ],
        scratch_types=[
            pltpu.VMEM((2, CH, D), jnp.float32),
            pltpu.VMEM((RB + 16,), jnp.int32),
            pltpu.VMEM((S * D,), jnp.float32),
            pltpu.VMEM((S * CW,), jnp.float32),
            pltpu.SemaphoreType.DMA,
            pltpu.SemaphoreType.DMA,
        ],
    )
    def seg(x_hbm, b_hbm, sums_hbm, cnts_hbm, rows, idxv, acc, cntv,
            sem0, sem1):
        cid = lax.axis_index("c")
        sid = lax.axis_index("s")
        wid = cid * NS + sid
        big = wid < BIGW
        nch = jnp.where(big, TB, TS)
        wbase = jnp.where(big, wid * RB, BIGW * RB + (wid - BIGW) * RS)

        zv = jnp.zeros((16,), jnp.float32)

        def zero_body(s, _):
            for j in range(DV):
                acc[pl.ds(s * D + 16 * j, 16)] = zv
            return 0

        lax.fori_loop(0, S, zero_body, 0)
        for k in range(S * CW // 16):
            cntv[pl.ds(16 * k, 16)] = zv

        @pl.when(big)
        def _():
            pltpu.sync_copy(b_hbm.at[pl.ds(wbase, RB)], idxv.at[pl.ds(0, RB)])

        @pl.when(jnp.logical_not(big))
        def _():
            pltpu.sync_copy(b_hbm.at[pl.ds(wbase, RS)], idxv.at[pl.ds(0, RS)])

        def issue(t):
            @pl.when(lax.rem(t, 2) == 0)
            def _():
                pltpu.async_copy(
                    x_hbm.at[pl.ds(wbase + t * CH, CH)],
                    rows.at[0], sem0)

            @pl.when(lax.rem(t, 2) == 1)
            def _():
                pltpu.async_copy(
                    x_hbm.at[pl.ds(wbase + t * CH, CH)],
                    rows.at[1], sem1)

        def wait_t(t):
            @pl.when(lax.rem(t, 2) == 0)
            def _():
                pltpu.make_async_copy(
                    x_hbm.at[pl.ds(wbase + t * CH, CH)],
                    rows.at[0], sem0).wait()

            @pl.when(lax.rem(t, 2) == 1)
            def _():
                pltpu.make_async_copy(
                    x_hbm.at[pl.ds(wbase + t * CH, CH)],
                    rows.at[1], sem1).wait()

        sixteen = jnp.full((CW,), 16.0, jnp.float32)
        one = jnp.ones((CW,), jnp.float32)

        def block_body(t, bk, _):
            g = t * CH + bk * 16
            tm = lax.rem(t, 2)
            row0 = bk * 16
            bv = idxv[pl.ds(g, 16)]
            uniform = bv[0] == bv[15]

            @pl.when(uniform)
            def _():
                s = bv[0]
                for j in range(DV):
                    a = rows[tm, row0, pl.ds(16 * j, 16)]
                    for r in range(1, 16):
                        a = a + rows[tm, row0 + r, pl.ds(16 * j, 16)]
                    plsc.addupdate(acc.at[pl.ds(s * D + 16 * j, 16)], a)
                plsc.addupdate(cntv.at[pl.ds(s * CW, CW)], sixteen)

            @pl.when(jnp.logical_not(uniform))
            def _():
                for r in range(16):
                    sr = bv[r]
                    for j in range(DV):
                        plsc.addupdate(
                            acc.at[pl.ds(sr * D + 16 * j, 16)],
                            rows[tm, row0 + r, pl.ds(16 * j, 16)])
                    plsc.addupdate(cntv.at[pl.ds(sr * CW, CW)], one)

            return 0

        def chunk_body(t, c):
            wait_t(t)

            @pl.when(t + 1 < nch)
            def _():
                issue(t + 1)

            return lax.fori_loop(0, NB, lambda bk, cc: block_body(t, bk, cc), c)

        issue(0)
        lax.fori_loop(0, nch, chunk_body, 0)

        pltpu.sync_copy(acc, sums_hbm.at[wid])
        pltpu.sync_copy(cntv, cnts_hbm.at[wid])

    return seg(xf, batch)


def _merge_kernel(s_ref, c_ref, o_ref):
    sums = jnp.sum(s_ref[...].reshape(NW, S, D), axis=0)
    cnt = jnp.sum(c_ref[...].reshape(NW, S, CW), axis=0)[:, 0:1]
    o_ref[...] = sums / jnp.maximum(cnt, 1.0)


def kernel(x, batch):
    sums, cnts = _sc_segment_sums(x, batch)
    return pl.pallas_call(
        _merge_kernel,
        out_shape=jax.ShapeDtypeStruct((S, D), jnp.float32),
    )(sums, cnts)
